# baseline (speedup 1.0000x reference)
import jax
import jax.numpy as jnp
from jax import lax
from jax.experimental import pallas as pl
from jax.experimental.pallas import tpu as pltpu
from jax.experimental.pallas import tpu_sc as plsc

PAD = 1
BEAM = 4
VOCAB = 100000
ROWS = 512
SENT = ROWS // BEAM
K = 8

NC, NS, L = 2, 16, 16
NW = NC * NS
NPANEL = ROWS // 8
PPW = NPANEL // NW

TIL = 128
NTFULL = VOCAB // TIL
TAIL0 = NTFULL * TIL
NTAIL = VOCAB - TAIL0

ST = 48
SLABC = ST * TIL
NSLAB = 16
ST2 = NTFULL - NSLAB * ST
SLABC2 = ST2 * TIL
NJ = NSLAB + 1
TMW = NJ * TIL

NEG = -1e30


def _tree(vals, op):
  vals = list(vals)
  while len(vals) > 1:
    nxt = [op(vals[i], vals[i + 1]) for i in range(0, len(vals) - 1, 2)]
    if len(vals) % 2:
      nxt.append(vals[-1])
    vals = nxt
  return vals[0]


def _scan_body(x_ref, tm_ref, ss_ref):
  j = pl.program_id(1)
  x = x_ref[...]
  col = lax.broadcasted_iota(jnp.int32, (8, SLABC), 1) + j * SLABC
  xm = jnp.where(col < VOCAB, x, NEG)
  ss_ref[...] = jnp.broadcast_to(
      jnp.sum(jnp.exp(xm), axis=1, keepdims=True), (8, TIL))
  parts = []
  for p in range(ST):
    tm = jnp.max(xm[:, p * TIL:(p + 1) * TIL], axis=1, keepdims=True)
    tm = jnp.where(j * ST + p >= NTFULL, NEG, tm)
    parts.append(tm)
  parts.append(jnp.full((8, TIL - ST), NEG))
  tm_ref[...] = jnp.concatenate(parts, axis=1)


def _stage1_body(logits, tmax, out_v, out_i,
                 buf_a, buf_b, tm_buf, tcb, st_v, st_i,
                 stg_v, stg_i, sem_a, sem_b, sem_t):
  cid = lax.axis_index("c")
  sid = lax.axis_index("s")
  wid = cid * NS + sid
  ii = lax.iota(jnp.int32, L)
  inf32 = jnp.float32(jnp.inf)

  bufs = (buf_a, buf_b)
  sems = (sem_a, sem_b)

  def merge(x, base, t, curv, curi):
    ni = base + ii
    snv, sni = plsc.sort_key_val(x, ni, descending=False)
    hv = jnp.maximum(curv, snv)
    hi = jnp.where(curv >= snv, curi, sni)
    shv, shi = plsc.sort_key_val(hv, hi, descending=True)
    curv = jnp.where(ii < K, shv, NEG)
    curi = shi
    t = jnp.min(jnp.where(ii < K, shv, inf32))
    return t, curv, curi

  NSPLIT = 8
  SPC = SLABC // NSPLIT

  def slab_start(panel, slab, which):
    row0 = panel * 8
    for k in range(NSPLIT):
      pltpu.async_copy(
          logits.at[pl.ds(row0, 8), pl.ds(slab * SLABC + k * SPC, SPC)],
          bufs[which].at[:, pl.ds(k * SPC, SPC)], sems[which])

  def slab_wait(panel, slab, which):
    row0 = panel * 8
    pltpu.make_async_copy(
        logits.at[pl.ds(row0, 8), pl.ds(slab * SLABC, SLABC)],
        bufs[which], sems[which]).wait()

  def tail_start(panel):
    row0 = panel * 8
    for k in range(ST2):
      pltpu.async_copy(
          logits.at[pl.ds(row0, 8), pl.ds(NSLAB * SLABC + k * TIL, TIL)],
          buf_a.at[:, pl.ds(k * TIL, TIL)], sems[0])

  def tail_wait(panel):
    row0 = panel * 8
    pltpu.make_async_copy(
        logits.at[pl.ds(row0, 8), pl.ds(NSLAB * SLABC, SLABC2)],
        buf_a.at[:, pl.ds(0, SLABC2)], sems[0]).wait()

  def process_slab(buf, si, is_first):
    if is_first is not False:
      @pl.when(is_first)
      def _():
        for r in range(8):
          v0 = buf[r, pl.ds(0, L)]
          buf[r, pl.ds(0, L)] = jnp.where(ii == PAD, NEG, v0)

    for r in range(8):
      curv = st_v[pl.ds(r * L, L)]
      curi = st_i[pl.ds(r * L, L)]
      tc_s = jnp.max(tcb[pl.ds(r * L, L)])
      t = jnp.min(jnp.where(ii < K, curv, inf32))
      m0 = tm_buf[r, pl.ds(si * TIL, L)]
      m1 = tm_buf[r, pl.ds(si * TIL + L, L)]
      m2 = tm_buf[r, pl.ds(si * TIL + 2 * L, L)]
      mm = jnp.maximum(m0, jnp.maximum(m1, m2))
      hit = jnp.any(mm >= jnp.maximum(tc_s, t))

      def noop(t, cv, ci):
        return (t, cv, ci)

      def rare(t, cv, ci):
        def q_body(q, st):
          t, cv, ci = st
          mv = tm_buf[r, pl.ds(si * TIL + q * L, L)]

          def w_cond(c):
            mv, t, cv, ci = c
            return jnp.any(mv >= jnp.maximum(tc_s, t))

          def w_body(c):
            mv, t, cv, ci = c
            gate = jnp.maximum(tc_s, t)
            lp = jnp.min(jnp.where(mv >= gate, ii, L))
            tl = q * L + lp

            def u_body(u, st):
              t, cv, ci = st
              x = buf[r, pl.ds(tl * TIL + u * L, L)]
              base = si * SLABC + tl * TIL + u * L

              def m1fn(t, cv, ci):
                return merge(x, base, t, cv, ci)

              t, cv, ci = lax.cond(
                  jnp.any(x >= jnp.maximum(tc_s, t)), m1fn, noop, t, cv, ci)
              return (t, cv, ci)

            t, cv, ci = lax.fori_loop(0, TIL // L, u_body, (t, cv, ci))
            mv = jnp.where(ii == lp, NEG, mv)
            return (mv, t, cv, ci)

          mv, t, cv, ci = lax.while_loop(w_cond, w_body, (mv, t, cv, ci))
          return (t, cv, ci)

        return lax.fori_loop(0, 3, q_body, (t, cv, ci))

      t, curv, curi = lax.cond(hit, rare, noop, t, curv, curi)
      st_v[pl.ds(r * L, L)] = curv
      st_i[pl.ds(r * L, L)] = curi

  def panel_body(pi, _):
    panel = wid * PPW + pi
    row0 = panel * 8
    pltpu.async_copy(tmax.at[pl.ds(row0, 8), :], tm_buf, sem_t).wait()
    for r in range(8):
      st_v[pl.ds(r * L, L)] = jnp.full((L,), NEG)
      st_i[pl.ds(r * L, L)] = jnp.zeros((L,), jnp.int32)

      def red(k, c, r=r):
        c0, c1, c2, c3 = c
        b = k * (4 * L)
        return (jnp.maximum(c0, tm_buf[r, pl.ds(b, L)]),
                jnp.maximum(c1, tm_buf[r, pl.ds(b + L, L)]),
                jnp.maximum(c2, tm_buf[r, pl.ds(b + 2 * L, L)]),
                jnp.maximum(c3, tm_buf[r, pl.ds(b + 3 * L, L)]))

      neg = jnp.full((L,), NEG)
      c = lax.fori_loop(0, TMW // (4 * L), red, (neg, neg, neg, neg))
      mlane = _tree(list(c), jnp.maximum)
      smx, _ = plsc.sort_key_val(mlane, ii, descending=True)
      tcv = jnp.min(jnp.where(ii < K, smx, jnp.float32(jnp.inf)))
      tcb[pl.ds(r * L, L)] = jnp.full((L,), tcv)

    slab_start(panel, 0, 0)

    def pair_body(i, _):
      slab_start(panel, 2 * i + 1, 1)
      slab_wait(panel, 2 * i, 0)
      process_slab(buf_a, 2 * i, i == 0)

      @pl.when(i < NSLAB // 2 - 1)
      def _():
        slab_start(panel, 2 * i + 2, 0)

      @pl.when(i == NSLAB // 2 - 1)
      def _():
        tail_start(panel)

      slab_wait(panel, 2 * i + 1, 1)
      process_slab(buf_b, 2 * i + 1, False)
      return 0

    lax.fori_loop(0, NSLAB // 2, pair_body, 0)
    tail_wait(panel)
    process_slab(buf_a, NSLAB, False)

    for r in range(8):
      stg_v[r, pl.ds(0, L)] = st_v[pl.ds(r * L, L)]
      stg_i[r, pl.ds(0, L)] = st_i[pl.ds(r * L, L)]
    pltpu.sync_copy(stg_v, out_v.at[pl.ds(row0, 8), :])
    pltpu.sync_copy(stg_i, out_i.at[pl.ds(row0, 8), :])
    return 0

  for r in range(8):
    for p in range(1, 8):
      stg_v[r, pl.ds(p * L, L)] = jnp.full((L,), NEG)
      stg_i[r, pl.ds(p * L, L)] = jnp.zeros((L,), jnp.int32)

  lax.fori_loop(0, PPW, panel_body, 0)


BL = 160


def _merge_body(v_ref, i_ref, ss_ref, osc_ref, obm_ref, otk_ref):
  v = v_ref[...]
  idx = i_ref[...]
  ss = ss_ref[...]
  lane = lax.broadcasted_iota(jnp.int32, (SENT, BEAM * BL), 1)
  sel = jnp.zeros((SENT, BEAM * BL), jnp.float32)
  for b in range(BEAM):
    s_b = jnp.sum(ss[:, b * NJ:(b + 1) * NJ], axis=1, keepdims=True)
    sel = sel + jnp.where(lane // BL == b, s_b, jnp.float32(0.0))
  score = v - jnp.log(sel)
  kidx = lax.broadcasted_iota(jnp.int32, (SENT, K), 1)
  osc = jnp.zeros((SENT, K), jnp.float32)
  obm = jnp.zeros((SENT, K), jnp.int32)
  otk = jnp.zeros((SENT, K), jnp.int32)
  for k in range(K):
    m = jnp.max(score, axis=1, keepdims=True)
    ism = score == m
    pos = jnp.min(jnp.where(ism, lane, BEAM * BL), axis=1, keepdims=True)
    onehot = lane == pos
    tok = jnp.sum(jnp.where(onehot, idx, 0), axis=1, keepdims=True)
    osc = jnp.where(kidx == k, m, osc)
    obm = jnp.where(kidx == k, pos // BL, obm)
    otk = jnp.where(kidx == k, tok, otk)
    score = jnp.where(onehot, NEG, score)
  osc_ref[...] = osc
  obm_ref[...] = obm
  otk_ref[...] = otk


@jax.jit
def kernel(logits):
  tmax, ssum = pl.pallas_call(
      _scan_body,
      grid=(ROWS // 8, NJ),
      in_specs=[pl.BlockSpec((8, SLABC), lambda i, j: (i, j))],
      out_specs=[pl.BlockSpec((8, TIL), lambda i, j: (i, j)),
                 pl.BlockSpec((8, TIL), lambda i, j: (i, j))],
      out_shape=(
          jax.ShapeDtypeStruct((ROWS, TMW), jnp.float32),
          jax.ShapeDtypeStruct((ROWS, TMW), jnp.float32),
      ),
  )(logits)

  mesh = plsc.VectorSubcoreMesh(core_axis_name="c", subcore_axis_name="s",
                                num_cores=NC, num_subcores=NS)
  stage1 = pl.kernel(
      _stage1_body,
      out_type=(
          jax.ShapeDtypeStruct((ROWS, 128), jnp.float32),
          jax.ShapeDtypeStruct((ROWS, 128), jnp.int32),
      ),
      mesh=mesh,
      compiler_params=pltpu.CompilerParams(needs_layout_passes=False),
      scratch_types=[
          pltpu.VMEM((8, SLABC), jnp.float32),
          pltpu.VMEM((8, SLABC), jnp.float32),
          pltpu.VMEM((8, TMW), jnp.float32),
          pltpu.VMEM((8 * L,), jnp.float32),
          pltpu.VMEM((8 * L,), jnp.float32),
          pltpu.VMEM((8 * L,), jnp.int32),
          pltpu.VMEM((8, 128), jnp.float32),
          pltpu.VMEM((8, 128), jnp.int32),
          pltpu.SemaphoreType.DMA,
          pltpu.SemaphoreType.DMA,
          pltpu.SemaphoreType.DMA,
      ],
  )
  tv, ti = stage1(logits, tmax)

  v2 = tv.reshape(SENT, BEAM * 128)
  i2 = ti.reshape(SENT, BEAM * 128)
  ss2 = ssum.reshape(SENT, BEAM, NJ, TIL)[..., 0].reshape(SENT, BEAM * NJ)
  tail = lax.slice(logits, (0, TAIL0), (ROWS, VOCAB)).reshape(
      SENT, BEAM * NTAIL)
  tidx = jnp.broadcast_to(
      TAIL0 + jnp.arange(NTAIL, dtype=jnp.int32), (SENT, NTAIL))
  vparts, iparts = [], []
  for b in range(BEAM):
    vparts += [v2[:, b * 128:(b + 1) * 128], tail[:, b * NTAIL:(b + 1) * NTAIL]]
    iparts += [i2[:, b * 128:(b + 1) * 128], tidx]
  comb_v = jnp.concatenate(vparts, axis=1)
  comb_i = jnp.concatenate(iparts, axis=1)
  scores, beams, toks = pl.pallas_call(
      _merge_body,
      out_shape=(
          jax.ShapeDtypeStruct((SENT, K), jnp.float32),
          jax.ShapeDtypeStruct((SENT, K), jnp.int32),
          jax.ShapeDtypeStruct((SENT, K), jnp.int32),
      ),
  )(comb_v, comb_i, ss2)
  return scores, beams, toks
